# baseline (speedup 1.0000x reference)
import jax
import jax.numpy as jnp
from jax.experimental import pallas as pl
from jax.experimental.pallas import tpu as pltpu

_N, _NFEAT, _NHID, _NCLASS = 4096, 128, 128, 64
_R = 256
_NBLK = _N // _R
_SPLIT = 1
_C = _N // _SPLIT


def _ln_relu(y, g, b):
    mu = jnp.mean(y, axis=-1, keepdims=True)
    d = y - mu
    var = jnp.mean(d * d, axis=-1, keepdims=True)
    return jnp.maximum(d * jax.lax.rsqrt(var + 1e-5) * g + b, 0.0)


def _gcn_body(*refs):
    adj_refs = refs[:_SPLIT]
    (x_ref, w1_ref, wnext_ref, bias_ref, lng_ref, lnb_ref, wo_ref, bo_ref,
     out_ref, z_a, z_b) = refs[_SPLIT:]
    l = pl.program_id(0)
    b = pl.program_id(1)

    @pl.when((l == 0) & (b == 0))
    def _():
        z_a[...] = jnp.dot(x_ref[...], w1_ref[...],
                           preferred_element_type=jnp.float32)

    g = lng_ref[0, :]
    beta = lnb_ref[0, :]

    def layer_h(z_ref):
        y = jnp.zeros((_R, _NHID), jnp.float32)
        for s in range(_SPLIT):
            a = adj_refs[s][0]
            y += jnp.dot(a, z_ref[pl.ds(s * _C, _C), :],
                         preferred_element_type=jnp.float32)
        return _ln_relu(y + bias_ref[0, 0, :], g, beta)

    @pl.when(l == 0)
    def _():
        h = layer_h(z_a)
        z_b[pl.ds(b * _R, _R), :] = jnp.dot(
            h, wnext_ref[0], preferred_element_type=jnp.float32)

    @pl.when(l == 1)
    def _():
        h = layer_h(z_b)
        z_a[pl.ds(b * _R, _R), :] = jnp.dot(
            h, wnext_ref[0], preferred_element_type=jnp.float32)

    @pl.when(l == 2)
    def _():
        h = layer_h(z_a)
        logits = jnp.dot(h, wo_ref[...],
                         preferred_element_type=jnp.float32) + bo_ref[0, :]
        m = jnp.max(logits, axis=-1, keepdims=True)
        e = jnp.exp(logits - m)
        s = jnp.sum(e, axis=-1, keepdims=True)
        out_ref[...] = logits - m - jnp.log(s)


def _adj_spec(s):
    return pl.BlockSpec((1, _R, _C), lambda l, b, _s=s: (l, b, _s))


def kernel(x, adj, W1, b1, W2, b2, W3, b3, ln_g, ln_b, Wo, bo):
    wnext = jnp.stack([W2, W3])
    bias = jnp.stack([b1, b2, b3])[:, None, :]
    lng = ln_g.reshape(1, _NHID)
    lnb = ln_b.reshape(1, _NHID)
    bo2 = bo.reshape(1, _NCLASS)

    return pl.pallas_call(
        _gcn_body,
        grid=(3, _NBLK),
        in_specs=[_adj_spec(s) for s in range(_SPLIT)] + [
            pl.BlockSpec((_N, _NFEAT), lambda l, b: (0, 0)),
            pl.BlockSpec((_NFEAT, _NHID), lambda l, b: (0, 0)),
            pl.BlockSpec((1, _NHID, _NHID),
                         lambda l, b: (jnp.minimum(l, 1), 0, 0)),
            pl.BlockSpec((1, 1, _NHID), lambda l, b: (l, 0, 0)),
            pl.BlockSpec((1, _NHID), lambda l, b: (0, 0)),
            pl.BlockSpec((1, _NHID), lambda l, b: (0, 0)),
            pl.BlockSpec((_NHID, _NCLASS), lambda l, b: (0, 0)),
            pl.BlockSpec((1, _NCLASS), lambda l, b: (0, 0)),
        ],
        out_specs=pl.BlockSpec((_R, _NCLASS), lambda l, b: (b, 0)),
        out_shape=jax.ShapeDtypeStruct((_N, _NCLASS), jnp.float32),
        scratch_shapes=[
            pltpu.VMEM((_N, _NHID), jnp.float32),
            pltpu.VMEM((_N, _NHID), jnp.float32),
        ],
        compiler_params=pltpu.CompilerParams(
            dimension_semantics=("arbitrary", "arbitrary")),
    )(*([adj] * _SPLIT), x, W1, wnext, bias, lng, lnb, Wo, bo2)


# speedup vs baseline: 1.2887x; 1.2887x over previous
import jax
import jax.numpy as jnp
from jax.experimental import pallas as pl
from jax.experimental.pallas import tpu as pltpu

_N, _NFEAT, _NHID, _NCLASS = 4096, 128, 128, 64
_R = 1024
_NBLK = _N // _R
_SPLIT = 1
_C = _N // _SPLIT


def _ln_relu(y, g, b):
    mu = jnp.mean(y, axis=-1, keepdims=True)
    d = y - mu
    var = jnp.mean(d * d, axis=-1, keepdims=True)
    return jnp.maximum(d * jax.lax.rsqrt(var + 1e-5) * g + b, 0.0)


def _gcn_body(*refs):
    adj_refs = refs[:_SPLIT]
    (x_ref, w1_ref, wnext_ref, bias_ref, lng_ref, lnb_ref, wo_ref, bo_ref,
     out_ref, z_a, z_b) = refs[_SPLIT:]
    l = pl.program_id(0)
    b = pl.program_id(1)

    @pl.when((l == 0) & (b == 0))
    def _():
        z_a[...] = jnp.dot(x_ref[...], w1_ref[...],
                           preferred_element_type=jnp.float32)

    g = lng_ref[0, :]
    beta = lnb_ref[0, :]

    def layer_h(z_ref):
        y = jnp.zeros((_R, _NHID), jnp.float32)
        for s in range(_SPLIT):
            a = adj_refs[s][0]
            y += jnp.dot(a, z_ref[pl.ds(s * _C, _C), :],
                         preferred_element_type=jnp.float32)
        return _ln_relu(y + bias_ref[0, 0, :], g, beta)

    @pl.when(l == 0)
    def _():
        h = layer_h(z_a)
        z_b[pl.ds(b * _R, _R), :] = jnp.dot(
            h, wnext_ref[0], preferred_element_type=jnp.float32)

    @pl.when(l == 1)
    def _():
        h = layer_h(z_b)
        z_a[pl.ds(b * _R, _R), :] = jnp.dot(
            h, wnext_ref[0], preferred_element_type=jnp.float32)

    @pl.when(l == 2)
    def _():
        h = layer_h(z_a)
        logits = jnp.dot(h, wo_ref[...],
                         preferred_element_type=jnp.float32) + bo_ref[0, :]
        m = jnp.max(logits, axis=-1, keepdims=True)
        e = jnp.exp(logits - m)
        s = jnp.sum(e, axis=-1, keepdims=True)
        out_ref[...] = logits - m - jnp.log(s)


def _adj_spec(s):
    return pl.BlockSpec((1, _R, _C), lambda l, b, _s=s: (l, b, _s))


def kernel(x, adj, W1, b1, W2, b2, W3, b3, ln_g, ln_b, Wo, bo):
    wnext = jnp.stack([W2, W3])
    bias = jnp.stack([b1, b2, b3])[:, None, :]
    lng = ln_g.reshape(1, _NHID)
    lnb = ln_b.reshape(1, _NHID)
    bo2 = bo.reshape(1, _NCLASS)

    return pl.pallas_call(
        _gcn_body,
        grid=(3, _NBLK),
        in_specs=[_adj_spec(s) for s in range(_SPLIT)] + [
            pl.BlockSpec((_N, _NFEAT), lambda l, b: (0, 0)),
            pl.BlockSpec((_NFEAT, _NHID), lambda l, b: (0, 0)),
            pl.BlockSpec((1, _NHID, _NHID),
                         lambda l, b: (jnp.minimum(l, 1), 0, 0)),
            pl.BlockSpec((1, 1, _NHID), lambda l, b: (l, 0, 0)),
            pl.BlockSpec((1, _NHID), lambda l, b: (0, 0)),
            pl.BlockSpec((1, _NHID), lambda l, b: (0, 0)),
            pl.BlockSpec((_NHID, _NCLASS), lambda l, b: (0, 0)),
            pl.BlockSpec((1, _NCLASS), lambda l, b: (0, 0)),
        ],
        out_specs=pl.BlockSpec((_R, _NCLASS), lambda l, b: (b, 0)),
        out_shape=jax.ShapeDtypeStruct((_N, _NCLASS), jnp.float32),
        scratch_shapes=[
            pltpu.VMEM((_N, _NHID), jnp.float32),
            pltpu.VMEM((_N, _NHID), jnp.float32),
        ],
        compiler_params=pltpu.CompilerParams(
            dimension_semantics=("arbitrary", "arbitrary")),
    )(*([adj] * _SPLIT), x, W1, wnext, bias, lng, lnb, Wo, bo2)
